# baseline (speedup 1.0000x reference)
import functools

import jax
import jax.numpy as jnp
from jax import lax
from jax.experimental import pallas as pl
from jax.experimental.pallas import tpu as pltpu
from jax.experimental.pallas import tpu_sc as plsc

_INFO = plsc.get_sparse_core_info()
_NC, _NS, _L = _INFO.num_cores, _INFO.num_subcores, _INFO.num_lanes
_NW = _NC * _NS

_ROWS = 512
_COLS = 512
_D = 256
_B = 16384
_BPW = _B // _NW
_CH = 128
_NCHUNK = _BPW // _CH


def _body(xf_hbm, a0_hbm, a1_hbm, out_hbm, i_v, j_v, idx_v, buf0, buf1, sem0, sem1):
    wid = lax.axis_index("s") * _NC + lax.axis_index("c")
    base = wid * _BPW

    pltpu.sync_copy(a0_hbm.at[pl.ds(base, _BPW)], i_v)
    pltpu.sync_copy(a1_hbm.at[pl.ds(base, _BPW)], j_v)

    for c in range(_NCHUNK):
        row = idx_v.at[c]
        for g in range(_CH // _L):
            s = pl.ds(c * _CH + g * _L, _L)
            row[pl.ds(g * _L, _L)] = i_v[s] * _COLS + j_v[s]

    bufs = (buf0, buf1)
    sems = (sem0, sem1)
    copies = [None] * _NCHUNK
    for c in range(min(2, _NCHUNK)):
        copies[c] = pltpu.async_copy(xf_hbm.at[idx_v.at[c]], bufs[c % 2], sems[c % 2])
    for c in range(_NCHUNK):
        copies[c].wait()
        pltpu.sync_copy(bufs[c % 2], out_hbm.at[pl.ds(base + c * _CH, _CH)])
        if c + 2 < _NCHUNK:
            copies[c + 2] = pltpu.async_copy(
                xf_hbm.at[idx_v.at[c + 2]], bufs[c % 2], sems[c % 2]
            )


_pooling_kernel = functools.partial(
    pl.kernel,
    mesh=plsc.VectorSubcoreMesh(core_axis_name="c", subcore_axis_name="s"),
    out_type=jax.ShapeDtypeStruct((_B, _D), jnp.float32),
    scratch_types=[
        pltpu.VMEM((_BPW,), jnp.int32),
        pltpu.VMEM((_BPW,), jnp.int32),
        pltpu.VMEM((_NCHUNK, _CH), jnp.int32),
        pltpu.VMEM((_CH, _D), jnp.float32),
        pltpu.VMEM((_CH, _D), jnp.float32),
        pltpu.SemaphoreType.DMA,
        pltpu.SemaphoreType.DMA,
    ],
)(_body)


def kernel(x, edge_index, agent_nodes):
    del edge_index
    xf = x.reshape(_ROWS * _COLS, _D)
    an = agent_nodes.astype(jnp.int32)
    return _pooling_kernel(xf, an[:, 0], an[:, 1])


# speedup vs baseline: 1.0205x; 1.0205x over previous
import functools

import jax
import jax.numpy as jnp
from jax import lax
from jax.experimental import pallas as pl
from jax.experimental.pallas import tpu as pltpu
from jax.experimental.pallas import tpu_sc as plsc

_INFO = plsc.get_sparse_core_info()
_NC, _NS, _L = _INFO.num_cores, _INFO.num_subcores, _INFO.num_lanes
_NW = _NC * _NS

_ROWS = 512
_COLS = 512
_D = 256
_B = 16384
_BPW = _B // _NW
_CH = 128
_NCHUNK = _BPW // _CH


_NBUF = 3


def _body(xf_hbm, a0_hbm, a1_hbm, out_hbm, i_v, j_v, idx_v,
          buf0, buf1, buf2, gsem0, gsem1, gsem2, wsem0, wsem1, wsem2):
    wid = lax.axis_index("s") * _NC + lax.axis_index("c")
    base = wid * _BPW

    pltpu.sync_copy(a0_hbm.at[pl.ds(base, _BPW)], i_v)
    pltpu.sync_copy(a1_hbm.at[pl.ds(base, _BPW)], j_v)

    for c in range(_NCHUNK):
        row = idx_v.at[c]
        for g in range(_CH // _L):
            s = pl.ds(c * _CH + g * _L, _L)
            row[pl.ds(g * _L, _L)] = i_v[s] * _COLS + j_v[s]

    bufs = (buf0, buf1, buf2)
    gsems = (gsem0, gsem1, gsem2)
    wsems = (wsem0, wsem1, wsem2)
    gcopies = [None] * _NCHUNK
    wcopies = [None] * _NCHUNK
    for c in range(min(_NBUF, _NCHUNK)):
        gcopies[c] = pltpu.async_copy(
            xf_hbm.at[idx_v.at[c]], bufs[c % _NBUF], gsems[c % _NBUF]
        )
    for c in range(_NCHUNK):
        gcopies[c].wait()
        wcopies[c] = pltpu.async_copy(
            bufs[c % _NBUF], out_hbm.at[pl.ds(base + c * _CH, _CH)], wsems[c % _NBUF]
        )
        nc = c + _NBUF
        if nc < _NCHUNK:
            wcopies[c].wait()
            gcopies[nc] = pltpu.async_copy(
                xf_hbm.at[idx_v.at[nc]], bufs[c % _NBUF], gsems[c % _NBUF]
            )
    for c in range(max(0, _NCHUNK - _NBUF), _NCHUNK):
        wcopies[c].wait()


_pooling_kernel = functools.partial(
    pl.kernel,
    mesh=plsc.VectorSubcoreMesh(core_axis_name="c", subcore_axis_name="s"),
    out_type=jax.ShapeDtypeStruct((_B, _D), jnp.float32),
    scratch_types=[
        pltpu.VMEM((_BPW,), jnp.int32),
        pltpu.VMEM((_BPW,), jnp.int32),
        pltpu.VMEM((_NCHUNK, _CH), jnp.int32),
        pltpu.VMEM((_CH, _D), jnp.float32),
        pltpu.VMEM((_CH, _D), jnp.float32),
        pltpu.VMEM((_CH, _D), jnp.float32),
        pltpu.SemaphoreType.DMA,
        pltpu.SemaphoreType.DMA,
        pltpu.SemaphoreType.DMA,
        pltpu.SemaphoreType.DMA,
        pltpu.SemaphoreType.DMA,
        pltpu.SemaphoreType.DMA,
    ],
)(_body)


def kernel(x, edge_index, agent_nodes):
    del edge_index
    xf = x.reshape(_ROWS * _COLS, _D)
    an = agent_nodes.astype(jnp.int32)
    return _pooling_kernel(xf, an[:, 0], an[:, 1])


# speedup vs baseline: 1.0470x; 1.0260x over previous
import functools

import jax
import jax.numpy as jnp
from jax import lax
from jax.experimental import pallas as pl
from jax.experimental.pallas import tpu as pltpu
from jax.experimental.pallas import tpu_sc as plsc

_INFO = plsc.get_sparse_core_info()
_NC, _NS, _L = _INFO.num_cores, _INFO.num_subcores, _INFO.num_lanes
_NW = _NC * _NS

_ROWS = 512
_COLS = 512
_D = 256
_B = 16384
_BPW = _B // _NW
_CH = 64
_NCHUNK = _BPW // _CH
_NBUF = 6


def _body(xf_hbm, a0_hbm, a1_hbm, out_hbm, *scratch):
    i_v, j_v, idx_v = scratch[0:3]
    bufs = scratch[3:3 + _NBUF]
    gsems = scratch[3 + _NBUF:3 + 2 * _NBUF]
    wsems = scratch[3 + 2 * _NBUF:3 + 3 * _NBUF]

    wid = lax.axis_index("s") * _NC + lax.axis_index("c")
    base = wid * _BPW

    pltpu.sync_copy(a0_hbm.at[pl.ds(base, _BPW)], i_v)
    pltpu.sync_copy(a1_hbm.at[pl.ds(base, _BPW)], j_v)

    for c in range(_NCHUNK):
        row = idx_v.at[c]
        for g in range(_CH // _L):
            s = pl.ds(c * _CH + g * _L, _L)
            row[pl.ds(g * _L, _L)] = i_v[s] * _COLS + j_v[s]

    gcopies = [None] * _NCHUNK
    wcopies = [None] * _NCHUNK
    for c in range(min(_NBUF, _NCHUNK)):
        gcopies[c] = pltpu.async_copy(
            xf_hbm.at[idx_v.at[c]], bufs[c % _NBUF], gsems[c % _NBUF]
        )
    for c in range(_NCHUNK):
        gcopies[c].wait()
        wcopies[c] = pltpu.async_copy(
            bufs[c % _NBUF], out_hbm.at[pl.ds(base + c * _CH, _CH)], wsems[c % _NBUF]
        )
        nc = c + _NBUF
        if nc < _NCHUNK:
            wcopies[c].wait()
            gcopies[nc] = pltpu.async_copy(
                xf_hbm.at[idx_v.at[nc]], bufs[c % _NBUF], gsems[c % _NBUF]
            )
    for c in range(max(0, _NCHUNK - _NBUF), _NCHUNK):
        wcopies[c].wait()


_pooling_kernel = functools.partial(
    pl.kernel,
    mesh=plsc.VectorSubcoreMesh(core_axis_name="c", subcore_axis_name="s"),
    out_type=jax.ShapeDtypeStruct((_B, _D), jnp.float32),
    scratch_types=[
        pltpu.VMEM((_BPW,), jnp.int32),
        pltpu.VMEM((_BPW,), jnp.int32),
        pltpu.VMEM((_NCHUNK, _CH), jnp.int32),
    ]
    + [pltpu.VMEM((_CH, _D), jnp.float32) for _ in range(_NBUF)]
    + [pltpu.SemaphoreType.DMA for _ in range(2 * _NBUF)],
)(_body)


def kernel(x, edge_index, agent_nodes):
    del edge_index
    xf = x.reshape(_ROWS * _COLS, _D)
    an = agent_nodes.astype(jnp.int32)
    return _pooling_kernel(xf, an[:, 0], an[:, 1])
